# baseline (speedup 1.0000x reference)
import jax
import jax.numpy as jnp
from jax import lax
from jax.experimental import pallas as pl
from jax.experimental.pallas import tpu as pltpu
from jax.experimental.pallas import tpu_sc as plsc

N_ROWS = 65536
N_UNARY = 128
N_B = 64
NC = 2
NS = 16
NW = NC * NS
RPW = N_ROWS // NW
BLK = 64
NBLK = RPW // BLK
NPAIR = NBLK // 2
IDX_CHUNK = 4096
NCHUNK = N_ROWS // IDX_CHUNK
L = 16
SCAN_UNROLL = 4


def _body(deltas, index1, index2, out, bout,
          s0, s1, winner1, winner2,
          idx1a, idx2a, idx1b, idx2b, d1a, d2a, d1b, d2b,
          g1a, g2a, g1b, g2b, oba, obb, bb,
          ss0, ss1, sg1a, sg2a, sg1b, sg2b, soa, sob, sbg, sbw):
  wid = lax.axis_index("s") * NC + lax.axis_index("c")
  base = wid * RPW
  iota = lax.iota(jnp.int32, L)
  neg1 = jnp.full((L,), -1, jnp.int32)
  zeros_f = jnp.zeros((L,), jnp.float32)
  base_vec = jnp.full((L,), base, jnp.int32)

  def init_body(j, _):
    for u in range(4):
      winner1[pl.ds((j * 4 + u) * L, L)] = neg1
      winner2[pl.ds((j * 4 + u) * L, L)] = neg1
    return 0
  lax.fori_loop(0, RPW // L // 4, init_body, 0)

  stages = [(s0, ss0), (s1, ss1)]
  total = 2 * NCHUNK
  cps = [None] * total
  cps[0] = pltpu.async_copy(index1.at[pl.ds(0, IDX_CHUNK)], s0, ss0)
  for ci in range(total):
    if ci + 1 < total:
      src = index1 if (ci + 1) < NCHUNK else index2
      off = ((ci + 1) % NCHUNK) * IDX_CHUNK
      nbuf, nsem = stages[(ci + 1) % 2]
      cps[ci + 1] = pltpu.async_copy(src.at[pl.ds(off, IDX_CHUNK)],
                                     nbuf, nsem)
    cps[ci].wait()
    buf, _ = stages[ci % 2]
    winner = winner1 if ci < NCHUNK else winner2
    gbase = (ci % NCHUNK) * IDX_CHUNK

    def scan_body(j, _, buf=buf, winner=winner, gbase=gbase):
      for u in range(SCAN_UNROLL):
        jj = j * SCAN_UNROLL + u
        v = buf[pl.ds(jj * L, L)]
        ivec = jnp.full((L,), gbase + jj * L, jnp.int32) + iota
        local = v - base_vec
        m = local.astype(jnp.uint32) < jnp.uint32(RPW)
        plsc.store_scatter(winner, [local], ivec, mask=m)
      return 0
    lax.fori_loop(0, IDX_CHUNK // L // SCAN_UNROLL, scan_body, 0)

  def build(k, idx1, idx2, dd1, dd2):
    rowbase = base + k * BLK

    def chunk_body(kk, carry):
      c1, c2 = carry
      off = k * BLK + kk * L
      rowid = jnp.full((L,), kk * L, jnp.int32) + iota
      self_row = jnp.full((L,), rowbase + kk * L, jnp.int32) + iota

      w1 = winner1[pl.ds(off, L)]
      dead1 = w1 < 0
      idx1[pl.ds(kk * L, L)] = jnp.where(dead1, self_row, w1)
      plsc.store_compressed(dd1.at[pl.ds(c1, L)], rowid, mask=dead1)
      c1 = c1 + plsc.all_reduce_population_count(dead1)[0]

      w2 = winner2[pl.ds(off, L)]
      dead2 = w2 < 0
      idx2[pl.ds(kk * L, L)] = jnp.where(dead2, self_row, w2)
      plsc.store_compressed(dd2.at[pl.ds(c2, L)], rowid, mask=dead2)
      c2 = c2 + plsc.all_reduce_population_count(dead2)[0]
      return (c1, c2)

    return lax.fori_loop(0, BLK // L, chunk_body,
                         (jnp.int32(0), jnp.int32(0)))

  def fire_gathers(idx1, idx2, g1, g2, sem1, sem2):
    pltpu.async_copy(deltas.at[idx1, pl.ds(0, N_UNARY)], g1, sem1)
    pltpu.async_copy(deltas.at[idx2, pl.ds(N_UNARY, N_UNARY)], g2, sem2)

  def wait_gathers(idx1, idx2, g1, g2, sem1, sem2):
    pltpu.make_async_copy(deltas.at[idx1, pl.ds(0, N_UNARY)], g1, sem1).wait()
    pltpu.make_async_copy(deltas.at[idx2, pl.ds(N_UNARY, N_UNARY)], g2,
                          sem2).wait()

  def wait_out(ob, sem):
    pltpu.make_async_copy(ob, out.at[pl.ds(base, BLK)], sem).wait()

  def fire_bgather(k):
    pltpu.async_copy(
        deltas.at[pl.ds(base + k * BLK, BLK), pl.ds(2 * N_UNARY, N_B)],
        bb, sbg)

  def wait_bgather():
    pltpu.make_async_copy(
        deltas.at[pl.ds(base, BLK), pl.ds(2 * N_UNARY, N_B)],
        bb, sbg).wait()

  def fire_bwrite(k):
    pltpu.async_copy(bb, bout.at[pl.ds(base + k * BLK, BLK)], sbw)

  def wait_bwrite():
    pltpu.make_async_copy(bb, bout.at[pl.ds(base, BLK)], sbw).wait()

  def zero_dead(dd, cnt, buf):
    def zbody(t, _):
      rows = dd[pl.ds(t * L, L)]
      zm = (jnp.full((L,), t * L, jnp.int32) + iota) < jnp.full(
          (L,), cnt, jnp.int32)
      for c in range(N_UNARY):
        col = jnp.full((L,), c, jnp.int32)
        plsc.store_scatter(buf, [rows, col], zeros_f, mask=zm)
      return 0
    lax.fori_loop(0, (cnt + L - 1) // L, zbody, 0)

  c1a, c2a = build(jnp.int32(0), idx1a, idx2a, d1a, d2a)
  fire_gathers(idx1a, idx2a, g1a, g2a, sg1a, sg2a)
  c1b, c2b = build(jnp.int32(1), idx1b, idx2b, d1b, d2b)
  fire_gathers(idx1b, idx2b, g1b, g2b, sg1b, sg2b)

  fire_bgather(jnp.int32(0))

  def pair_body(t, carry):
    c1a, c2a, c1b, c2b = carry
    kA = 2 * t
    kB = 2 * t + 1

    wait_gathers(idx1a, idx2a, g1a, g2a, sg1a, sg2a)
    zero_dead(d1a, c1a, g1a)
    zero_dead(d2a, c2a, g2a)

    @pl.when(t > 0)
    def _():
      wait_out(oba, soa)

    def add_a(r):
      for cc in range(N_UNARY // L):
        s = pl.ds(cc * L, L)
        oba[r, s] = g1a[r, s] + g2a[r, s]
    plsc.parallel_loop(0, BLK, 1, unroll=4)(add_a)
    pltpu.async_copy(oba, out.at[pl.ds(base + kA * BLK, BLK)], soa)

    kA2 = lax.rem(kA + 2, NBLK)
    c1a2, c2a2 = build(kA2, idx1a, idx2a, d1a, d2a)
    fire_gathers(idx1a, idx2a, g1a, g2a, sg1a, sg2a)

    wait_bgather()
    fire_bwrite(kA)
    wait_bwrite()
    fire_bgather(kB)

    wait_gathers(idx1b, idx2b, g1b, g2b, sg1b, sg2b)
    zero_dead(d1b, c1b, g1b)
    zero_dead(d2b, c2b, g2b)

    @pl.when(t > 0)
    def _():
      wait_out(obb, sob)

    def add_b(r):
      for cc in range(N_UNARY // L):
        s = pl.ds(cc * L, L)
        obb[r, s] = g1b[r, s] + g2b[r, s]
    plsc.parallel_loop(0, BLK, 1, unroll=4)(add_b)
    pltpu.async_copy(obb, out.at[pl.ds(base + kB * BLK, BLK)], sob)

    kB2 = lax.rem(kB + 2, NBLK)
    c1b2, c2b2 = build(kB2, idx1b, idx2b, d1b, d2b)
    fire_gathers(idx1b, idx2b, g1b, g2b, sg1b, sg2b)

    wait_bgather()
    fire_bwrite(kB)
    wait_bwrite()
    fire_bgather(lax.rem(kB + 1, NBLK))

    return (c1a2, c2a2, c1b2, c2b2)

  lax.fori_loop(0, NPAIR, pair_body, (c1a, c2a, c1b, c2b))

  wait_gathers(idx1a, idx2a, g1a, g2a, sg1a, sg2a)
  wait_gathers(idx1b, idx2b, g1b, g2b, sg1b, sg2b)
  wait_bgather()
  wait_out(oba, soa)
  wait_out(obb, sob)


def kernel(unary, deltas, index1, index2):
  del unary
  mesh = plsc.VectorSubcoreMesh(core_axis_name="c", subcore_axis_name="s")
  f = pl.kernel(
      _body,
      out_type=(
          jax.ShapeDtypeStruct((N_ROWS, N_UNARY), jnp.float32),
          jax.ShapeDtypeStruct((N_ROWS, N_B), jnp.float32),
      ),
      mesh=mesh,
      compiler_params=pltpu.CompilerParams(needs_layout_passes=False),
      scratch_types=[
          pltpu.VMEM((IDX_CHUNK,), jnp.int32),
          pltpu.VMEM((IDX_CHUNK,), jnp.int32),
          pltpu.VMEM((RPW,), jnp.int32),
          pltpu.VMEM((RPW,), jnp.int32),
          pltpu.VMEM((BLK,), jnp.int32),
          pltpu.VMEM((BLK,), jnp.int32),
          pltpu.VMEM((BLK,), jnp.int32),
          pltpu.VMEM((BLK,), jnp.int32),
          pltpu.VMEM((BLK + L,), jnp.int32),
          pltpu.VMEM((BLK + L,), jnp.int32),
          pltpu.VMEM((BLK + L,), jnp.int32),
          pltpu.VMEM((BLK + L,), jnp.int32),
          pltpu.VMEM((BLK, N_UNARY), jnp.float32),
          pltpu.VMEM((BLK, N_UNARY), jnp.float32),
          pltpu.VMEM((BLK, N_UNARY), jnp.float32),
          pltpu.VMEM((BLK, N_UNARY), jnp.float32),
          pltpu.VMEM((BLK, N_UNARY), jnp.float32),
          pltpu.VMEM((BLK, N_UNARY), jnp.float32),
          pltpu.VMEM((BLK, N_B), jnp.float32),
          pltpu.SemaphoreType.DMA,
          pltpu.SemaphoreType.DMA,
          pltpu.SemaphoreType.DMA,
          pltpu.SemaphoreType.DMA,
          pltpu.SemaphoreType.DMA,
          pltpu.SemaphoreType.DMA,
          pltpu.SemaphoreType.DMA,
          pltpu.SemaphoreType.DMA,
          pltpu.SemaphoreType.DMA,
          pltpu.SemaphoreType.DMA,
      ],
  )
  return f(deltas, index1, index2)


# speedup vs baseline: 1.0258x; 1.0258x over previous
import jax
import jax.numpy as jnp
from jax import lax
from jax.experimental import pallas as pl
from jax.experimental.pallas import tpu as pltpu
from jax.experimental.pallas import tpu_sc as plsc

N_ROWS = 65536
N_UNARY = 128
N_B = 64
NC = 2
NS = 16
NW = NC * NS
RPW = N_ROWS // NW
BLK = 64
NBLK = RPW // BLK
NPAIR = NBLK // 2
IDX_CHUNK = 4096
NCHUNK = N_ROWS // IDX_CHUNK
L = 16
SCAN_UNROLL = 4


def _body(deltas, index1, index2, out, bout,
          s0, s1, winner1, winner2,
          idx1a, idx2a, idx1b, idx2b, d1a, d2a, d1b, d2b,
          g1a, g2a, g1b, g2b, oba, obb, bba, bbb,
          ss0, ss1, sg1a, sg2a, sg1b, sg2b, soa, sob,
          sbga, sbgb, sbwa, sbwb):
  wid = lax.axis_index("s") * NC + lax.axis_index("c")
  base = wid * RPW
  iota = lax.iota(jnp.int32, L)
  neg1 = jnp.full((L,), -1, jnp.int32)
  zeros_f = jnp.zeros((L,), jnp.float32)
  base_vec = jnp.full((L,), base, jnp.int32)

  def init_body(j, _):
    for u in range(4):
      winner1[pl.ds((j * 4 + u) * L, L)] = neg1
      winner2[pl.ds((j * 4 + u) * L, L)] = neg1
    return 0
  lax.fori_loop(0, RPW // L // 4, init_body, 0)

  stages = [(s0, ss0), (s1, ss1)]
  total = 2 * NCHUNK
  cps = [None] * total
  cps[0] = pltpu.async_copy(index1.at[pl.ds(0, IDX_CHUNK)], s0, ss0)
  for ci in range(total):
    if ci + 1 < total:
      src = index1 if (ci + 1) < NCHUNK else index2
      off = ((ci + 1) % NCHUNK) * IDX_CHUNK
      nbuf, nsem = stages[(ci + 1) % 2]
      cps[ci + 1] = pltpu.async_copy(src.at[pl.ds(off, IDX_CHUNK)],
                                     nbuf, nsem)
    cps[ci].wait()
    buf, _ = stages[ci % 2]
    winner = winner1 if ci < NCHUNK else winner2
    gbase = (ci % NCHUNK) * IDX_CHUNK

    def scan_body(j, _, buf=buf, winner=winner, gbase=gbase):
      for u in range(SCAN_UNROLL):
        jj = j * SCAN_UNROLL + u
        v = buf[pl.ds(jj * L, L)]
        ivec = jnp.full((L,), gbase + jj * L, jnp.int32) + iota
        local = v - base_vec
        m = local.astype(jnp.uint32) < jnp.uint32(RPW)
        plsc.store_scatter(winner, [local], ivec, mask=m)
      return 0
    lax.fori_loop(0, IDX_CHUNK // L // SCAN_UNROLL, scan_body, 0)

  def build(k, idx1, idx2, dd1, dd2):
    rowbase = base + k * BLK

    def chunk_body(kk, carry):
      c1, c2 = carry
      off = k * BLK + kk * L
      rowid = jnp.full((L,), kk * L, jnp.int32) + iota
      self_row = jnp.full((L,), rowbase + kk * L, jnp.int32) + iota

      w1 = winner1[pl.ds(off, L)]
      dead1 = w1 < 0
      idx1[pl.ds(kk * L, L)] = jnp.where(dead1, self_row, w1)
      plsc.store_compressed(dd1.at[pl.ds(c1, L)], rowid, mask=dead1)
      c1 = c1 + plsc.all_reduce_population_count(dead1)[0]

      w2 = winner2[pl.ds(off, L)]
      dead2 = w2 < 0
      idx2[pl.ds(kk * L, L)] = jnp.where(dead2, self_row, w2)
      plsc.store_compressed(dd2.at[pl.ds(c2, L)], rowid, mask=dead2)
      c2 = c2 + plsc.all_reduce_population_count(dead2)[0]
      return (c1, c2)

    return lax.fori_loop(0, BLK // L, chunk_body,
                         (jnp.int32(0), jnp.int32(0)))

  def fire_gathers(idx1, idx2, g1, g2, sem1, sem2):
    pltpu.async_copy(deltas.at[idx1, pl.ds(0, N_UNARY)], g1, sem1)
    pltpu.async_copy(deltas.at[idx2, pl.ds(N_UNARY, N_UNARY)], g2, sem2)

  def wait_gathers(idx1, idx2, g1, g2, sem1, sem2):
    pltpu.make_async_copy(deltas.at[idx1, pl.ds(0, N_UNARY)], g1, sem1).wait()
    pltpu.make_async_copy(deltas.at[idx2, pl.ds(N_UNARY, N_UNARY)], g2,
                          sem2).wait()

  def wait_out(ob, sem):
    pltpu.make_async_copy(ob, out.at[pl.ds(base, BLK)], sem).wait()

  def fire_bgather(k, buf, sem):
    pltpu.async_copy(
        deltas.at[pl.ds(base + k * BLK, BLK), pl.ds(2 * N_UNARY, N_B)],
        buf, sem)

  def wait_bgather(buf, sem):
    pltpu.make_async_copy(
        deltas.at[pl.ds(base, BLK), pl.ds(2 * N_UNARY, N_B)],
        buf, sem).wait()

  def fire_bwrite(k, buf, sem):
    pltpu.async_copy(buf, bout.at[pl.ds(base + k * BLK, BLK)], sem)

  def wait_bwrite(buf, sem):
    pltpu.make_async_copy(buf, bout.at[pl.ds(base, BLK)], sem).wait()

  def zero_dead(dd, cnt, buf):
    def zbody(t, _):
      rows = dd[pl.ds(t * L, L)]
      zm = (jnp.full((L,), t * L, jnp.int32) + iota) < jnp.full(
          (L,), cnt, jnp.int32)
      for c in range(N_UNARY):
        col = jnp.full((L,), c, jnp.int32)
        plsc.store_scatter(buf, [rows, col], zeros_f, mask=zm)
      return 0
    lax.fori_loop(0, (cnt + L - 1) // L, zbody, 0)

  c1a, c2a = build(jnp.int32(0), idx1a, idx2a, d1a, d2a)
  fire_gathers(idx1a, idx2a, g1a, g2a, sg1a, sg2a)
  c1b, c2b = build(jnp.int32(1), idx1b, idx2b, d1b, d2b)
  fire_gathers(idx1b, idx2b, g1b, g2b, sg1b, sg2b)

  fire_bgather(jnp.int32(0), bba, sbga)
  fire_bgather(jnp.int32(1), bbb, sbgb)

  def pair_body(t, carry):
    c1a, c2a, c1b, c2b = carry
    kA = 2 * t
    kB = 2 * t + 1

    wait_gathers(idx1a, idx2a, g1a, g2a, sg1a, sg2a)
    zero_dead(d1a, c1a, g1a)
    zero_dead(d2a, c2a, g2a)

    @pl.when(t > 0)
    def _():
      wait_out(oba, soa)

    def add_a(r):
      for cc in range(N_UNARY // L):
        s = pl.ds(cc * L, L)
        oba[r, s] = g1a[r, s] + g2a[r, s]
    plsc.parallel_loop(0, BLK, 1, unroll=4)(add_a)
    pltpu.async_copy(oba, out.at[pl.ds(base + kA * BLK, BLK)], soa)

    kA2 = lax.rem(kA + 2, NBLK)
    c1a2, c2a2 = build(kA2, idx1a, idx2a, d1a, d2a)
    fire_gathers(idx1a, idx2a, g1a, g2a, sg1a, sg2a)

    wait_bgather(bba, sbga)
    fire_bwrite(kA, bba, sbwa)

    wait_gathers(idx1b, idx2b, g1b, g2b, sg1b, sg2b)
    zero_dead(d1b, c1b, g1b)
    zero_dead(d2b, c2b, g2b)

    @pl.when(t > 0)
    def _():
      wait_out(obb, sob)

    def add_b(r):
      for cc in range(N_UNARY // L):
        s = pl.ds(cc * L, L)
        obb[r, s] = g1b[r, s] + g2b[r, s]
    plsc.parallel_loop(0, BLK, 1, unroll=4)(add_b)
    pltpu.async_copy(obb, out.at[pl.ds(base + kB * BLK, BLK)], sob)

    kB2 = lax.rem(kB + 2, NBLK)
    c1b2, c2b2 = build(kB2, idx1b, idx2b, d1b, d2b)
    fire_gathers(idx1b, idx2b, g1b, g2b, sg1b, sg2b)

    wait_bgather(bbb, sbgb)
    fire_bwrite(kB, bbb, sbwb)
    wait_bwrite(bba, sbwa)
    fire_bgather(lax.rem(kA + 2, NBLK), bba, sbga)
    wait_bwrite(bbb, sbwb)
    fire_bgather(lax.rem(kB + 2, NBLK), bbb, sbgb)

    return (c1a2, c2a2, c1b2, c2b2)

  lax.fori_loop(0, NPAIR, pair_body, (c1a, c2a, c1b, c2b))

  wait_gathers(idx1a, idx2a, g1a, g2a, sg1a, sg2a)
  wait_gathers(idx1b, idx2b, g1b, g2b, sg1b, sg2b)
  wait_bgather(bba, sbga)
  wait_bgather(bbb, sbgb)
  wait_out(oba, soa)
  wait_out(obb, sob)


def kernel(unary, deltas, index1, index2):
  del unary
  mesh = plsc.VectorSubcoreMesh(core_axis_name="c", subcore_axis_name="s")
  f = pl.kernel(
      _body,
      out_type=(
          jax.ShapeDtypeStruct((N_ROWS, N_UNARY), jnp.float32),
          jax.ShapeDtypeStruct((N_ROWS, N_B), jnp.float32),
      ),
      mesh=mesh,
      compiler_params=pltpu.CompilerParams(needs_layout_passes=False),
      scratch_types=[
          pltpu.VMEM((IDX_CHUNK,), jnp.int32),
          pltpu.VMEM((IDX_CHUNK,), jnp.int32),
          pltpu.VMEM((RPW,), jnp.int32),
          pltpu.VMEM((RPW,), jnp.int32),
          pltpu.VMEM((BLK,), jnp.int32),
          pltpu.VMEM((BLK,), jnp.int32),
          pltpu.VMEM((BLK,), jnp.int32),
          pltpu.VMEM((BLK,), jnp.int32),
          pltpu.VMEM((BLK + L,), jnp.int32),
          pltpu.VMEM((BLK + L,), jnp.int32),
          pltpu.VMEM((BLK + L,), jnp.int32),
          pltpu.VMEM((BLK + L,), jnp.int32),
          pltpu.VMEM((BLK, N_UNARY), jnp.float32),
          pltpu.VMEM((BLK, N_UNARY), jnp.float32),
          pltpu.VMEM((BLK, N_UNARY), jnp.float32),
          pltpu.VMEM((BLK, N_UNARY), jnp.float32),
          pltpu.VMEM((BLK, N_UNARY), jnp.float32),
          pltpu.VMEM((BLK, N_UNARY), jnp.float32),
          pltpu.VMEM((BLK, N_B), jnp.float32),
          pltpu.VMEM((BLK, N_B), jnp.float32),
          pltpu.SemaphoreType.DMA,
          pltpu.SemaphoreType.DMA,
          pltpu.SemaphoreType.DMA,
          pltpu.SemaphoreType.DMA,
          pltpu.SemaphoreType.DMA,
          pltpu.SemaphoreType.DMA,
          pltpu.SemaphoreType.DMA,
          pltpu.SemaphoreType.DMA,
          pltpu.SemaphoreType.DMA,
          pltpu.SemaphoreType.DMA,
          pltpu.SemaphoreType.DMA,
          pltpu.SemaphoreType.DMA,
      ],
  )
  return f(deltas, index1, index2)


# speedup vs baseline: 1.1027x; 1.0750x over previous
import jax
import jax.numpy as jnp
from jax import lax
from jax.experimental import pallas as pl
from jax.experimental.pallas import tpu as pltpu
from jax.experimental.pallas import tpu_sc as plsc

N_ROWS = 65536
N_UNARY = 128
N_B = 64
NC = 2
NS = 16
NW = NC * NS
RPW = N_ROWS // NW
BLK = 128
NBLK = RPW // BLK
NPAIR = NBLK // 2
IDX_CHUNK = 4096
NCHUNK = N_ROWS // IDX_CHUNK
L = 16
SCAN_UNROLL = 4


def _body(deltas, index1, index2, out,
          s0, s1, winner1, winner2,
          idx1a, idx2a, idx1b, idx2b, d1a, d2a, d1b, d2b,
          g1a, g2a, g1b, g2b, oba, obb,
          ss0, ss1, sg1a, sg2a, sg1b, sg2b, soa, sob):
  wid = lax.axis_index("s") * NC + lax.axis_index("c")
  base = wid * RPW
  iota = lax.iota(jnp.int32, L)
  neg1 = jnp.full((L,), -1, jnp.int32)
  zeros_f = jnp.zeros((L,), jnp.float32)
  base_vec = jnp.full((L,), base, jnp.int32)

  def init_body(j, _):
    for u in range(4):
      winner1[pl.ds((j * 4 + u) * L, L)] = neg1
      winner2[pl.ds((j * 4 + u) * L, L)] = neg1
    return 0
  lax.fori_loop(0, RPW // L // 4, init_body, 0)

  stages = [(s0, ss0), (s1, ss1)]
  total = 2 * NCHUNK
  cps = [None] * total
  cps[0] = pltpu.async_copy(index1.at[pl.ds(0, IDX_CHUNK)], s0, ss0)
  for ci in range(total):
    if ci + 1 < total:
      src = index1 if (ci + 1) < NCHUNK else index2
      off = ((ci + 1) % NCHUNK) * IDX_CHUNK
      nbuf, nsem = stages[(ci + 1) % 2]
      cps[ci + 1] = pltpu.async_copy(src.at[pl.ds(off, IDX_CHUNK)],
                                     nbuf, nsem)
    cps[ci].wait()
    buf, _ = stages[ci % 2]
    winner = winner1 if ci < NCHUNK else winner2
    gbase = (ci % NCHUNK) * IDX_CHUNK

    def scan_body(j, _, buf=buf, winner=winner, gbase=gbase):
      for u in range(SCAN_UNROLL):
        jj = j * SCAN_UNROLL + u
        v = buf[pl.ds(jj * L, L)]
        ivec = jnp.full((L,), gbase + jj * L, jnp.int32) + iota
        local = v - base_vec
        m = local.astype(jnp.uint32) < jnp.uint32(RPW)
        plsc.store_scatter(winner, [local], ivec, mask=m)
      return 0
    lax.fori_loop(0, IDX_CHUNK // L // SCAN_UNROLL, scan_body, 0)

  def build(k, idx1, idx2, dd1, dd2):
    rowbase = base + k * BLK

    def chunk_body(kk, carry):
      c1, c2 = carry
      off = k * BLK + kk * L
      rowid = jnp.full((L,), kk * L, jnp.int32) + iota
      self_row = jnp.full((L,), rowbase + kk * L, jnp.int32) + iota

      w1 = winner1[pl.ds(off, L)]
      dead1 = w1 < 0
      idx1[pl.ds(kk * L, L)] = jnp.where(dead1, self_row, w1)
      plsc.store_compressed(dd1.at[pl.ds(c1, L)], rowid, mask=dead1)
      c1 = c1 + plsc.all_reduce_population_count(dead1)[0]

      w2 = winner2[pl.ds(off, L)]
      dead2 = w2 < 0
      idx2[pl.ds(kk * L, L)] = jnp.where(dead2, self_row, w2)
      plsc.store_compressed(dd2.at[pl.ds(c2, L)], rowid, mask=dead2)
      c2 = c2 + plsc.all_reduce_population_count(dead2)[0]
      return (c1, c2)

    return lax.fori_loop(0, BLK // L, chunk_body,
                         (jnp.int32(0), jnp.int32(0)))

  def fire_gathers(idx1, idx2, g1, g2, sem1, sem2):
    pltpu.async_copy(deltas.at[idx1, pl.ds(0, N_UNARY)], g1, sem1)
    pltpu.async_copy(deltas.at[idx2, pl.ds(N_UNARY, N_UNARY)], g2, sem2)

  def wait_gathers(idx1, idx2, g1, g2, sem1, sem2):
    pltpu.make_async_copy(deltas.at[idx1, pl.ds(0, N_UNARY)], g1, sem1).wait()
    pltpu.make_async_copy(deltas.at[idx2, pl.ds(N_UNARY, N_UNARY)], g2,
                          sem2).wait()

  def wait_out(ob, sem):
    pltpu.make_async_copy(ob, out.at[pl.ds(base, BLK)], sem).wait()

  def zero_dead(dd, cnt, buf):
    def zbody(t, _):
      rows = dd[pl.ds(t * L, L)]
      zm = (jnp.full((L,), t * L, jnp.int32) + iota) < jnp.full(
          (L,), cnt, jnp.int32)
      for c in range(N_UNARY):
        col = jnp.full((L,), c, jnp.int32)
        plsc.store_scatter(buf, [rows, col], zeros_f, mask=zm)
      return 0
    lax.fori_loop(0, (cnt + L - 1) // L, zbody, 0)

  c1a, c2a = build(jnp.int32(0), idx1a, idx2a, d1a, d2a)
  fire_gathers(idx1a, idx2a, g1a, g2a, sg1a, sg2a)
  c1b, c2b = build(jnp.int32(1), idx1b, idx2b, d1b, d2b)
  fire_gathers(idx1b, idx2b, g1b, g2b, sg1b, sg2b)

  def pair_body(t, carry):
    c1a, c2a, c1b, c2b = carry
    kA = 2 * t
    kB = 2 * t + 1

    wait_gathers(idx1a, idx2a, g1a, g2a, sg1a, sg2a)
    zero_dead(d1a, c1a, g1a)
    zero_dead(d2a, c2a, g2a)

    @pl.when(t > 0)
    def _():
      wait_out(oba, soa)

    def add_a(r):
      for cc in range(N_UNARY // L):
        s = pl.ds(cc * L, L)
        oba[r, s] = g1a[r, s] + g2a[r, s]
    plsc.parallel_loop(0, BLK, 1, unroll=4)(add_a)
    pltpu.async_copy(oba, out.at[pl.ds(base + kA * BLK, BLK)], soa)

    kA2 = lax.rem(kA + 2, NBLK)
    c1a2, c2a2 = build(kA2, idx1a, idx2a, d1a, d2a)
    fire_gathers(idx1a, idx2a, g1a, g2a, sg1a, sg2a)

    wait_gathers(idx1b, idx2b, g1b, g2b, sg1b, sg2b)
    zero_dead(d1b, c1b, g1b)
    zero_dead(d2b, c2b, g2b)

    @pl.when(t > 0)
    def _():
      wait_out(obb, sob)

    def add_b(r):
      for cc in range(N_UNARY // L):
        s = pl.ds(cc * L, L)
        obb[r, s] = g1b[r, s] + g2b[r, s]
    plsc.parallel_loop(0, BLK, 1, unroll=4)(add_b)
    pltpu.async_copy(obb, out.at[pl.ds(base + kB * BLK, BLK)], sob)

    kB2 = lax.rem(kB + 2, NBLK)
    c1b2, c2b2 = build(kB2, idx1b, idx2b, d1b, d2b)
    fire_gathers(idx1b, idx2b, g1b, g2b, sg1b, sg2b)

    return (c1a2, c2a2, c1b2, c2b2)

  lax.fori_loop(0, NPAIR, pair_body, (c1a, c2a, c1b, c2b))

  wait_gathers(idx1a, idx2a, g1a, g2a, sg1a, sg2a)
  wait_gathers(idx1b, idx2b, g1b, g2b, sg1b, sg2b)
  wait_out(oba, soa)
  wait_out(obb, sob)


def _b_copy_body(d_ref, o_ref):
  o_ref[...] = d_ref[:, :N_B]


def kernel(unary, deltas, index1, index2):
  del unary
  mesh = plsc.VectorSubcoreMesh(core_axis_name="c", subcore_axis_name="s")
  f = pl.kernel(
      _body,
      out_type=jax.ShapeDtypeStruct((N_ROWS, N_UNARY), jnp.float32),
      mesh=mesh,
      compiler_params=pltpu.CompilerParams(needs_layout_passes=False),
      cost_estimate=pl.CostEstimate(
          flops=70_000_000, transcendentals=0, bytes_accessed=135_000_000),
      scratch_types=[
          pltpu.VMEM((IDX_CHUNK,), jnp.int32),
          pltpu.VMEM((IDX_CHUNK,), jnp.int32),
          pltpu.VMEM((RPW,), jnp.int32),
          pltpu.VMEM((RPW,), jnp.int32),
          pltpu.VMEM((BLK,), jnp.int32),
          pltpu.VMEM((BLK,), jnp.int32),
          pltpu.VMEM((BLK,), jnp.int32),
          pltpu.VMEM((BLK,), jnp.int32),
          pltpu.VMEM((BLK + L,), jnp.int32),
          pltpu.VMEM((BLK + L,), jnp.int32),
          pltpu.VMEM((BLK + L,), jnp.int32),
          pltpu.VMEM((BLK + L,), jnp.int32),
          pltpu.VMEM((BLK, N_UNARY), jnp.float32),
          pltpu.VMEM((BLK, N_UNARY), jnp.float32),
          pltpu.VMEM((BLK, N_UNARY), jnp.float32),
          pltpu.VMEM((BLK, N_UNARY), jnp.float32),
          pltpu.VMEM((BLK, N_UNARY), jnp.float32),
          pltpu.VMEM((BLK, N_UNARY), jnp.float32),
          pltpu.SemaphoreType.DMA,
          pltpu.SemaphoreType.DMA,
          pltpu.SemaphoreType.DMA,
          pltpu.SemaphoreType.DMA,
          pltpu.SemaphoreType.DMA,
          pltpu.SemaphoreType.DMA,
          pltpu.SemaphoreType.DMA,
          pltpu.SemaphoreType.DMA,
      ],
  )
  bout = pl.pallas_call(
      _b_copy_body,
      grid=(64,),
      in_specs=[pl.BlockSpec((1024, N_UNARY), lambda i: (i, 2))],
      out_specs=pl.BlockSpec((1024, N_B), lambda i: (i, 0)),
      out_shape=jax.ShapeDtypeStruct((N_ROWS, N_B), jnp.float32),
  )(deltas)
  out1 = f(deltas, index1, index2)
  return (out1, bout)


# speedup vs baseline: 1.1043x; 1.0015x over previous
import jax
import jax.numpy as jnp
from jax import lax
from jax.experimental import pallas as pl
from jax.experimental.pallas import tpu as pltpu
from jax.experimental.pallas import tpu_sc as plsc

N_ROWS = 65536
N_UNARY = 128
N_B = 64
NC = 2
NS = 16
NW = NC * NS
RPW = N_ROWS // NW
BLK = 128
NBLK = RPW // BLK
NPAIR = NBLK // 2
IDX_CHUNK = 4096
NCHUNK = N_ROWS // IDX_CHUNK
L = 16
SCAN_UNROLL = 4


def _body(deltas, index1, index2, out,
          s0, s1, winner1, winner2,
          idx1a, idx2a, idx1b, idx2b, d1a, d2a, d1b, d2b,
          g1a, g2a, g1b, g2b, oba, obb,
          ss0, ss1, sg1a, sg2a, sg1b, sg2b, soa, sob):
  wid = lax.axis_index("s") * NC + lax.axis_index("c")
  base = wid * RPW
  iota = lax.iota(jnp.int32, L)
  neg1 = jnp.full((L,), -1, jnp.int32)
  zeros_f = jnp.zeros((L,), jnp.float32)
  base_vec = jnp.full((L,), base, jnp.int32)

  def init_body(j, _):
    for u in range(4):
      winner1[pl.ds((j * 4 + u) * L, L)] = neg1
      winner2[pl.ds((j * 4 + u) * L, L)] = neg1
    return 0
  lax.fori_loop(0, RPW // L // 4, init_body, 0)

  stages = [(s0, ss0), (s1, ss1)]
  total = 2 * NCHUNK
  cps = [None] * total
  cps[0] = pltpu.async_copy(index1.at[pl.ds(0, IDX_CHUNK)], s0, ss0)
  for ci in range(total):
    if ci + 1 < total:
      src = index1 if (ci + 1) < NCHUNK else index2
      off = ((ci + 1) % NCHUNK) * IDX_CHUNK
      nbuf, nsem = stages[(ci + 1) % 2]
      cps[ci + 1] = pltpu.async_copy(src.at[pl.ds(off, IDX_CHUNK)],
                                     nbuf, nsem)
    cps[ci].wait()
    buf, _ = stages[ci % 2]
    winner = winner1 if ci < NCHUNK else winner2
    gbase = (ci % NCHUNK) * IDX_CHUNK

    def scan_body(j, _, buf=buf, winner=winner, gbase=gbase):
      for u in range(SCAN_UNROLL):
        jj = j * SCAN_UNROLL + u
        v = buf[pl.ds(jj * L, L)]
        ivec = jnp.full((L,), gbase + jj * L, jnp.int32) + iota
        local = v - base_vec
        m = local.astype(jnp.uint32) < jnp.uint32(RPW)
        plsc.store_scatter(winner, [local], ivec, mask=m)
      return 0
    lax.fori_loop(0, IDX_CHUNK // L // SCAN_UNROLL, scan_body, 0)

  def build(k, idx1, idx2, dd1, dd2):
    rowbase = base + k * BLK

    def chunk_body(kk, carry):
      c1, c2 = carry
      off = k * BLK + kk * L
      rowid = jnp.full((L,), kk * L, jnp.int32) + iota
      self_row = jnp.full((L,), rowbase + kk * L, jnp.int32) + iota

      w1 = winner1[pl.ds(off, L)]
      dead1 = w1 < 0
      idx1[pl.ds(kk * L, L)] = jnp.where(dead1, self_row, w1)
      plsc.store_compressed(dd1.at[pl.ds(c1, L)], rowid, mask=dead1)
      c1 = c1 + plsc.all_reduce_population_count(dead1)[0]

      w2 = winner2[pl.ds(off, L)]
      dead2 = w2 < 0
      idx2[pl.ds(kk * L, L)] = jnp.where(dead2, self_row, w2)
      plsc.store_compressed(dd2.at[pl.ds(c2, L)], rowid, mask=dead2)
      c2 = c2 + plsc.all_reduce_population_count(dead2)[0]
      return (c1, c2)

    return lax.fori_loop(0, BLK // L, chunk_body,
                         (jnp.int32(0), jnp.int32(0)))

  def fire_gathers(idx1, idx2, g1, g2, sem1, sem2):
    pltpu.async_copy(deltas.at[idx1, pl.ds(0, N_UNARY)], g1, sem1)
    pltpu.async_copy(deltas.at[idx2, pl.ds(N_UNARY, N_UNARY)], g2, sem2)

  def wait_gathers(idx1, idx2, g1, g2, sem1, sem2):
    pltpu.make_async_copy(deltas.at[idx1, pl.ds(0, N_UNARY)], g1, sem1).wait()
    pltpu.make_async_copy(deltas.at[idx2, pl.ds(N_UNARY, N_UNARY)], g2,
                          sem2).wait()

  def wait_out(ob, sem):
    pltpu.make_async_copy(ob, out.at[pl.ds(base, BLK)], sem).wait()

  def zero_dead(dd, cnt, buf):
    def zbody(t, _):
      rows = dd[pl.ds(t * L, L)]
      zm = (jnp.full((L,), t * L, jnp.int32) + iota) < jnp.full(
          (L,), cnt, jnp.int32)
      for c in range(N_UNARY):
        col = jnp.full((L,), c, jnp.int32)
        plsc.store_scatter(buf, [rows, col], zeros_f, mask=zm)
      return 0
    lax.fori_loop(0, (cnt + L - 1) // L, zbody, 0)

  c1a, c2a = build(jnp.int32(0), idx1a, idx2a, d1a, d2a)
  fire_gathers(idx1a, idx2a, g1a, g2a, sg1a, sg2a)
  c1b, c2b = build(jnp.int32(1), idx1b, idx2b, d1b, d2b)
  fire_gathers(idx1b, idx2b, g1b, g2b, sg1b, sg2b)

  def pair_body(t, carry):
    c1a, c2a, c1b, c2b = carry
    kA = 2 * t
    kB = 2 * t + 1

    wait_gathers(idx1a, idx2a, g1a, g2a, sg1a, sg2a)
    zero_dead(d1a, c1a, g1a)
    zero_dead(d2a, c2a, g2a)

    @pl.when(t > 0)
    def _():
      wait_out(oba, soa)

    def add_a(r):
      for cc in range(N_UNARY // L):
        s = pl.ds(cc * L, L)
        oba[r, s] = g1a[r, s] + g2a[r, s]
    plsc.parallel_loop(0, BLK, 1, unroll=4)(add_a)
    pltpu.async_copy(oba, out.at[pl.ds(base + kA * BLK, BLK)], soa)

    kA2 = lax.rem(kA + 2, NBLK)
    c1a2, c2a2 = build(kA2, idx1a, idx2a, d1a, d2a)
    fire_gathers(idx1a, idx2a, g1a, g2a, sg1a, sg2a)

    wait_gathers(idx1b, idx2b, g1b, g2b, sg1b, sg2b)
    zero_dead(d1b, c1b, g1b)
    zero_dead(d2b, c2b, g2b)

    @pl.when(t > 0)
    def _():
      wait_out(obb, sob)

    def add_b(r):
      for cc in range(N_UNARY // L):
        s = pl.ds(cc * L, L)
        obb[r, s] = g1b[r, s] + g2b[r, s]
    plsc.parallel_loop(0, BLK, 1, unroll=4)(add_b)
    pltpu.async_copy(obb, out.at[pl.ds(base + kB * BLK, BLK)], sob)

    kB2 = lax.rem(kB + 2, NBLK)
    c1b2, c2b2 = build(kB2, idx1b, idx2b, d1b, d2b)
    fire_gathers(idx1b, idx2b, g1b, g2b, sg1b, sg2b)

    return (c1a2, c2a2, c1b2, c2b2)

  lax.fori_loop(0, NPAIR, pair_body, (c1a, c2a, c1b, c2b))

  wait_gathers(idx1a, idx2a, g1a, g2a, sg1a, sg2a)
  wait_gathers(idx1b, idx2b, g1b, g2b, sg1b, sg2b)
  wait_out(oba, soa)
  wait_out(obb, sob)


def _b_copy_body(d_ref, o_ref):
  o_ref[...] = d_ref[:, :N_B]


def kernel(unary, deltas, index1, index2):
  del unary
  mesh = plsc.VectorSubcoreMesh(core_axis_name="c", subcore_axis_name="s")
  f = pl.kernel(
      _body,
      out_type=jax.ShapeDtypeStruct((N_ROWS, N_UNARY), jnp.float32),
      mesh=mesh,
      compiler_params=pltpu.CompilerParams(needs_layout_passes=False,
                                           use_tc_tiling_on_sc=True),
      cost_estimate=pl.CostEstimate(
          flops=70_000_000, transcendentals=0, bytes_accessed=135_000_000),
      scratch_types=[
          pltpu.VMEM((IDX_CHUNK,), jnp.int32),
          pltpu.VMEM((IDX_CHUNK,), jnp.int32),
          pltpu.VMEM((RPW,), jnp.int32),
          pltpu.VMEM((RPW,), jnp.int32),
          pltpu.VMEM((BLK,), jnp.int32),
          pltpu.VMEM((BLK,), jnp.int32),
          pltpu.VMEM((BLK,), jnp.int32),
          pltpu.VMEM((BLK,), jnp.int32),
          pltpu.VMEM((BLK + L,), jnp.int32),
          pltpu.VMEM((BLK + L,), jnp.int32),
          pltpu.VMEM((BLK + L,), jnp.int32),
          pltpu.VMEM((BLK + L,), jnp.int32),
          pltpu.VMEM((BLK, N_UNARY), jnp.float32),
          pltpu.VMEM((BLK, N_UNARY), jnp.float32),
          pltpu.VMEM((BLK, N_UNARY), jnp.float32),
          pltpu.VMEM((BLK, N_UNARY), jnp.float32),
          pltpu.VMEM((BLK, N_UNARY), jnp.float32),
          pltpu.VMEM((BLK, N_UNARY), jnp.float32),
          pltpu.SemaphoreType.DMA,
          pltpu.SemaphoreType.DMA,
          pltpu.SemaphoreType.DMA,
          pltpu.SemaphoreType.DMA,
          pltpu.SemaphoreType.DMA,
          pltpu.SemaphoreType.DMA,
          pltpu.SemaphoreType.DMA,
          pltpu.SemaphoreType.DMA,
      ],
  )
  bout = pl.pallas_call(
      _b_copy_body,
      grid=(64,),
      in_specs=[pl.BlockSpec((1024, N_UNARY), lambda i: (i, 2))],
      out_specs=pl.BlockSpec((1024, N_B), lambda i: (i, 0)),
      out_shape=jax.ShapeDtypeStruct((N_ROWS, N_B), jnp.float32),
  )(deltas)
  out1 = f(deltas, index1, index2)
  return (out1, bout)


# speedup vs baseline: 1.3170x; 1.1926x over previous
import jax
import jax.numpy as jnp
from jax import lax
from jax.experimental import pallas as pl
from jax.experimental.pallas import tpu as pltpu
from jax.experimental.pallas import tpu_sc as plsc

N_ROWS = 65536
N_UNARY = 128
N_B = 64
NC = 2
NS = 16
NW = NC * NS
RPW = N_ROWS // NW
BLK = 128
NBLK = RPW // BLK
NPAIR = NBLK // 2
IDX_CHUNK = 4096
NCHUNK = N_ROWS // IDX_CHUNK
L = 16
SCAN_UNROLL = 4


def _scan_body(index1, index2, w1out, w2out,
               s0, s1, winner1, winner2, ss0, ss1):
  wid = lax.axis_index("s") * NC + lax.axis_index("c")
  base = wid * RPW
  iota = lax.iota(jnp.int32, L)
  neg1 = jnp.full((L,), -1, jnp.int32)
  base_vec = jnp.full((L,), base, jnp.int32)

  def init_body(j, _):
    for u in range(4):
      winner1[pl.ds((j * 4 + u) * L, L)] = neg1
      winner2[pl.ds((j * 4 + u) * L, L)] = neg1
    return 0
  lax.fori_loop(0, RPW // L // 4, init_body, 0)

  stages = [(s0, ss0), (s1, ss1)]
  total = 2 * NCHUNK
  cps = [None] * total
  cps[0] = pltpu.async_copy(index1.at[pl.ds(0, IDX_CHUNK)], s0, ss0)
  for ci in range(total):
    if ci + 1 < total:
      src = index1 if (ci + 1) < NCHUNK else index2
      off = ((ci + 1) % NCHUNK) * IDX_CHUNK
      nbuf, nsem = stages[(ci + 1) % 2]
      cps[ci + 1] = pltpu.async_copy(src.at[pl.ds(off, IDX_CHUNK)],
                                     nbuf, nsem)
    cps[ci].wait()
    buf, _ = stages[ci % 2]
    winner = winner1 if ci < NCHUNK else winner2
    gbase = (ci % NCHUNK) * IDX_CHUNK

    def scan_body(j, _, buf=buf, winner=winner, gbase=gbase):
      for u in range(SCAN_UNROLL):
        jj = j * SCAN_UNROLL + u
        v = buf[pl.ds(jj * L, L)]
        ivec = jnp.full((L,), gbase + jj * L, jnp.int32) + iota
        local = v - base_vec
        m = local.astype(jnp.uint32) < jnp.uint32(RPW)
        plsc.store_scatter(winner, [local], ivec, mask=m)
      return 0
    lax.fori_loop(0, IDX_CHUNK // L // SCAN_UNROLL, scan_body, 0)

  pltpu.sync_copy(winner1, w1out.at[pl.ds(base, RPW)])
  pltpu.sync_copy(winner2, w2out.at[pl.ds(base, RPW)])


def _gather_body(deltas, w1in, w2in, out,
                 winner1, winner2,
                 idx1a, idx2a, idx1b, idx2b, d1a, d2a, d1b, d2b,
                 g1a, g2a, g1b, g2b, oba, obb,
                 sg1a, sg2a, sg1b, sg2b, soa, sob):
  wid = lax.axis_index("s") * NC + lax.axis_index("c")
  base = wid * RPW
  iota = lax.iota(jnp.int32, L)
  zeros_f = jnp.zeros((L,), jnp.float32)
  pltpu.sync_copy(w1in.at[pl.ds(base, RPW)], winner1)
  pltpu.sync_copy(w2in.at[pl.ds(base, RPW)], winner2)

  def build(k, idx1, idx2, dd1, dd2):
    rowbase = base + k * BLK

    def chunk_body(kk, carry):
      c1, c2 = carry
      off = k * BLK + kk * L
      rowid = jnp.full((L,), kk * L, jnp.int32) + iota
      self_row = jnp.full((L,), rowbase + kk * L, jnp.int32) + iota

      w1 = winner1[pl.ds(off, L)]
      dead1 = w1 < 0
      idx1[pl.ds(kk * L, L)] = jnp.where(dead1, self_row, w1)
      plsc.store_compressed(dd1.at[pl.ds(c1, L)], rowid, mask=dead1)
      c1 = c1 + plsc.all_reduce_population_count(dead1)[0]

      w2 = winner2[pl.ds(off, L)]
      dead2 = w2 < 0
      idx2[pl.ds(kk * L, L)] = jnp.where(dead2, self_row, w2)
      plsc.store_compressed(dd2.at[pl.ds(c2, L)], rowid, mask=dead2)
      c2 = c2 + plsc.all_reduce_population_count(dead2)[0]
      return (c1, c2)

    return lax.fori_loop(0, BLK // L, chunk_body,
                         (jnp.int32(0), jnp.int32(0)))

  def fire_gathers(idx1, idx2, g1, g2, sem1, sem2):
    pltpu.async_copy(deltas.at[idx1, pl.ds(0, N_UNARY)], g1, sem1)
    pltpu.async_copy(deltas.at[idx2, pl.ds(N_UNARY, N_UNARY)], g2, sem2)

  def wait_gathers(idx1, idx2, g1, g2, sem1, sem2):
    pltpu.make_async_copy(deltas.at[idx1, pl.ds(0, N_UNARY)], g1, sem1).wait()
    pltpu.make_async_copy(deltas.at[idx2, pl.ds(N_UNARY, N_UNARY)], g2,
                          sem2).wait()

  def wait_out(ob, sem):
    pltpu.make_async_copy(ob, out.at[pl.ds(base, BLK)], sem).wait()

  def zero_dead(dd, cnt, buf):
    def zbody(t, _):
      rows = dd[pl.ds(t * L, L)]
      zm = (jnp.full((L,), t * L, jnp.int32) + iota) < jnp.full(
          (L,), cnt, jnp.int32)
      for c in range(N_UNARY):
        col = jnp.full((L,), c, jnp.int32)
        plsc.store_scatter(buf, [rows, col], zeros_f, mask=zm)
      return 0
    lax.fori_loop(0, (cnt + L - 1) // L, zbody, 0)

  c1a, c2a = build(jnp.int32(0), idx1a, idx2a, d1a, d2a)
  fire_gathers(idx1a, idx2a, g1a, g2a, sg1a, sg2a)
  c1b, c2b = build(jnp.int32(1), idx1b, idx2b, d1b, d2b)
  fire_gathers(idx1b, idx2b, g1b, g2b, sg1b, sg2b)

  def pair_body(t, carry):
    c1a, c2a, c1b, c2b = carry
    kA = 2 * t
    kB = 2 * t + 1

    wait_gathers(idx1a, idx2a, g1a, g2a, sg1a, sg2a)
    zero_dead(d1a, c1a, g1a)
    zero_dead(d2a, c2a, g2a)

    @pl.when(t > 0)
    def _():
      wait_out(oba, soa)

    def add_a(r):
      for cc in range(N_UNARY // L):
        s = pl.ds(cc * L, L)
        oba[r, s] = g1a[r, s] + g2a[r, s]
    plsc.parallel_loop(0, BLK, 1, unroll=4)(add_a)
    pltpu.async_copy(oba, out.at[pl.ds(base + kA * BLK, BLK)], soa)

    kA2 = lax.rem(kA + 2, NBLK)
    c1a2, c2a2 = build(kA2, idx1a, idx2a, d1a, d2a)
    fire_gathers(idx1a, idx2a, g1a, g2a, sg1a, sg2a)

    wait_gathers(idx1b, idx2b, g1b, g2b, sg1b, sg2b)
    zero_dead(d1b, c1b, g1b)
    zero_dead(d2b, c2b, g2b)

    @pl.when(t > 0)
    def _():
      wait_out(obb, sob)

    def add_b(r):
      for cc in range(N_UNARY // L):
        s = pl.ds(cc * L, L)
        obb[r, s] = g1b[r, s] + g2b[r, s]
    plsc.parallel_loop(0, BLK, 1, unroll=4)(add_b)
    pltpu.async_copy(obb, out.at[pl.ds(base + kB * BLK, BLK)], sob)

    kB2 = lax.rem(kB + 2, NBLK)
    c1b2, c2b2 = build(kB2, idx1b, idx2b, d1b, d2b)
    fire_gathers(idx1b, idx2b, g1b, g2b, sg1b, sg2b)

    return (c1a2, c2a2, c1b2, c2b2)

  lax.fori_loop(0, NPAIR, pair_body, (c1a, c2a, c1b, c2b))

  wait_gathers(idx1a, idx2a, g1a, g2a, sg1a, sg2a)
  wait_gathers(idx1b, idx2b, g1b, g2b, sg1b, sg2b)
  wait_out(oba, soa)
  wait_out(obb, sob)


def _b_copy_body(d_ref, o_ref):
  o_ref[...] = d_ref[:, :N_B]


def kernel(unary, deltas, index1, index2):
  del unary
  mesh = plsc.VectorSubcoreMesh(core_axis_name="c", subcore_axis_name="s")
  scan = pl.kernel(
      _scan_body,
      out_type=(
          jax.ShapeDtypeStruct((N_ROWS,), jnp.int32),
          jax.ShapeDtypeStruct((N_ROWS,), jnp.int32),
      ),
      mesh=mesh,
      compiler_params=pltpu.CompilerParams(needs_layout_passes=False),
      cost_estimate=pl.CostEstimate(
          flops=10_000_000, transcendentals=0, bytes_accessed=17_000_000),
      scratch_types=[
          pltpu.VMEM((IDX_CHUNK,), jnp.int32),
          pltpu.VMEM((IDX_CHUNK,), jnp.int32),
          pltpu.VMEM((RPW,), jnp.int32),
          pltpu.VMEM((RPW,), jnp.int32),
          pltpu.SemaphoreType.DMA,
          pltpu.SemaphoreType.DMA,
      ],
  )
  gather = pl.kernel(
      _gather_body,
      out_type=jax.ShapeDtypeStruct((N_ROWS, N_UNARY), jnp.float32),
      mesh=mesh,
      compiler_params=pltpu.CompilerParams(needs_layout_passes=False),
      cost_estimate=pl.CostEstimate(
          flops=50_000_000, transcendentals=0, bytes_accessed=100_000_000),
      scratch_types=[
          pltpu.VMEM((RPW,), jnp.int32),
          pltpu.VMEM((RPW,), jnp.int32),
          pltpu.VMEM((BLK,), jnp.int32),
          pltpu.VMEM((BLK,), jnp.int32),
          pltpu.VMEM((BLK,), jnp.int32),
          pltpu.VMEM((BLK,), jnp.int32),
          pltpu.VMEM((BLK + L,), jnp.int32),
          pltpu.VMEM((BLK + L,), jnp.int32),
          pltpu.VMEM((BLK + L,), jnp.int32),
          pltpu.VMEM((BLK + L,), jnp.int32),
          pltpu.VMEM((BLK, N_UNARY), jnp.float32),
          pltpu.VMEM((BLK, N_UNARY), jnp.float32),
          pltpu.VMEM((BLK, N_UNARY), jnp.float32),
          pltpu.VMEM((BLK, N_UNARY), jnp.float32),
          pltpu.VMEM((BLK, N_UNARY), jnp.float32),
          pltpu.VMEM((BLK, N_UNARY), jnp.float32),
          pltpu.SemaphoreType.DMA,
          pltpu.SemaphoreType.DMA,
          pltpu.SemaphoreType.DMA,
          pltpu.SemaphoreType.DMA,
          pltpu.SemaphoreType.DMA,
          pltpu.SemaphoreType.DMA,
      ],
  )
  w1, w2 = scan(index1, index2)
  bout = pl.pallas_call(
      _b_copy_body,
      grid=(64,),
      in_specs=[pl.BlockSpec((1024, N_UNARY), lambda i: (i, 2))],
      out_specs=pl.BlockSpec((1024, N_B), lambda i: (i, 0)),
      out_shape=jax.ShapeDtypeStruct((N_ROWS, N_B), jnp.float32),
  )(deltas)
  out1 = gather(deltas, w1, w2)
  return (out1, bout)
